# baseline (speedup 1.0000x reference)
import functools

import jax
import jax.numpy as jnp
from jax import lax
from jax.experimental import pallas as pl
from jax.experimental.pallas import tpu as pltpu
from jax.experimental.pallas import tpu_sc as plsc

N_CORES = 2
N_SUBCORES = 16
LANES = 16

_GATHER_DN = lax.GatherDimensionNumbers(
    offset_dims=(), collapsed_slice_dims=(0,), start_index_map=(0,))
_UNROLL = 4


def _make_sc_edge_pass(n_nodes, n_edges, msg_w, chunk):
    row_w = msg_w + LANES
    expand = msg_w == 64
    n_workers = N_CORES * N_SUBCORES
    e_per_w = n_edges // n_workers
    assert e_per_w * n_workers == n_edges
    n_chunks = -(-e_per_w // chunk)
    valid_last = e_per_w - (n_chunks - 1) * chunk
    assert valid_last % LANES == 0
    rows_per_tile = -(-n_nodes // (N_SUBCORES * chunk)) * chunk
    n_pad = rows_per_tile * N_SUBCORES
    n_copy_full = rows_per_tile // chunk
    assert rows_per_tile % 8 == 0 and n_pad > n_nodes

    mesh = plsc.VectorSubcoreMesh(core_axis_name="c", subcore_axis_name="s")

    assert n_chunks >= 5 and (n_chunks - 4) % 3 == 0

    @functools.partial(
        pl.kernel,
        out_type=jax.ShapeDtypeStruct((N_CORES * n_pad, row_w), jnp.float32),
        mesh=mesh,
        compiler_params=pltpu.CompilerParams(use_tc_tiling_on_sc=False),
        scratch_types=(
            [pltpu.VMEM((2, chunk), jnp.int32)] * 3
            + [pltpu.VMEM((chunk,), jnp.int32)] * 3
            + [pltpu.VMEM((chunk, row_w), jnp.float32)] * 3
            + [pltpu.VMEM((chunk, LANES), jnp.float32)] * 3
            + [pltpu.VMEM_SHARED((n_pad, row_w), jnp.float32)]
            + [pltpu.SemaphoreType.DMA] * 12
        ),
    )
    def sc_pass(srctab, dsttab, ei_hbm, out_hbm,
                ij0, ij1, ij2,
                dscat0, dscat1, dscat2, A0, A1, A2, B0, B1, B2, acc,
                ssi0, ssi1, ssi2,
                sga0, sga1, sga2, sgb0, sgb1, sgb2, sss0, sss1, sss2):
        c = lax.axis_index("c")
        s = lax.axis_index("s")
        wid = c * N_SUBCORES + s

        ij = (ij0, ij1, ij2)
        dscat = (dscat0, dscat1, dscat2)
        A = (A0, A1, A2)
        B = (B0, B1, B2)
        ssi = (ssi0, ssi1, ssi2)
        sga = (sga0, sga1, sga2)
        sgb = (sgb0, sgb1, sgb2)
        sss = (sss0, sss1, sss2)

        zeros16 = jnp.zeros((LANES,), jnp.float32)

        def zero_row(e, carry):
            for q in range(row_w // LANES):
                A0[e, pl.ds(q * LANES, LANES)] = zeros16
            return carry

        lax.fori_loop(0, chunk, zero_row, 0)
        r0 = s * rows_per_tile
        zdescs = []
        for zi in range(n_copy_full):
            zdescs.append(pltpu.async_copy(
                A0, acc.at[pl.ds(r0 + zi * chunk, chunk)], sss0))
        for zd in zdescs:
            zd.wait()
        plsc.subcore_barrier()

        iota = lax.iota(jnp.int32, LANES)
        colpat = jnp.right_shift(iota, 3)

        ebase0 = wid * e_per_w

        def issue_idx(m, b):
            eb = ebase0 + m * chunk
            pltpu.async_copy(ei_hbm.at[:, pl.ds(eb, chunk)], ij[b], ssi[b])

        def wait_idx(b):
            pltpu.make_async_copy(
                ei_hbm.at[:, pl.ds(0, chunk)], ij[b], ssi[b]).wait()

        def issue_gather(b):
            pltpu.async_copy(srctab.at[ij[b].at[0]], A[b], sga[b])
            pltpu.async_copy(dsttab.at[ij[b].at[1]], B[b], sgb[b])

        def wait_gather(b):
            pltpu.make_async_copy(srctab.at[ij[b].at[0]], A[b], sga[b]).wait()
            pltpu.make_async_copy(dsttab.at[ij[b].at[1]], B[b], sgb[b]).wait()

        def issue_scatter(b):
            pltpu.async_copy(A[b], acc.at[dscat[b]], sss[b], add=True)

        def wait_scatter(b):
            pltpu.make_async_copy(A[b], acc.at[dscat[b]], sss[b]).wait()

        def compute(b):
            Ab, Bb = A[b], B[b]

            def one_edge(e):
                t = Ab[e, pl.ds(msg_w, LANES)] + Bb[e, pl.ds(0, LANES)]
                w = jnp.exp(jnp.maximum(t, t * 0.2))
                Ab[e, pl.ds(msg_w, LANES)] = w
                if expand:
                    for j in range(msg_w // LANES):
                        wj = lax.gather(
                            w, (colpat + (2 * j)).reshape(LANES, 1),
                            _GATHER_DN, slice_sizes=(1,),
                            mode=lax.GatherScatterMode.PROMISE_IN_BOUNDS)
                        Ab[e, pl.ds(LANES * j, LANES)] = (
                            Ab[e, pl.ds(LANES * j, LANES)] * wj)
                else:
                    for j in range(msg_w // LANES):
                        Ab[e, pl.ds(LANES * j, LANES)] = (
                            Ab[e, pl.ds(LANES * j, LANES)] * w)

            def edge_body(eg, icarry):
                for u in range(_UNROLL):
                    one_edge(eg * _UNROLL + u)
                return icarry

            lax.fori_loop(0, chunk // _UNROLL, edge_body, 0)

        def copy_dscat(b, last=False):
            for q in range(chunk // LANES):
                if last and q * LANES >= valid_last:
                    dscat[b][pl.ds(q * LANES, LANES)] = jnp.full(
                        (LANES,), n_pad - 1, jnp.int32)
                else:
                    dscat[b][pl.ds(q * LANES, LANES)] = (
                        ij[b][1, pl.ds(q * LANES, LANES)])

        def steady_step(m, b, first=False, wait_next=True, issue_next=True,
                        last=False):
            m = jnp.asarray(m, jnp.int32)
            bn = (b + 1) % 3
            if not first:
                wait_scatter(bn)
            if wait_next:
                wait_idx(bn)
                issue_gather(bn)
            wait_gather(b)
            copy_dscat(b, last=last)
            if issue_next:
                issue_idx(m + 2, (b + 2) % 3)
            compute(b)
            issue_scatter(b)

        issue_idx(0, 0)
        issue_idx(1, 1)
        wait_idx(0)
        issue_gather(0)
        steady_step(0, 0, first=True)
        steady_step(1, 1, first=True)

        def triple_body(tp, carry):
            steady_step(3 * tp + 2, 2)
            steady_step(3 * tp + 3, 0)
            steady_step(3 * tp + 4, 1)
            return carry

        n_triples = (n_chunks - 4) // 3
        lax.fori_loop(0, n_triples, triple_body, 0)

        mt = n_chunks - 2
        steady_step(mt, mt % 3, issue_next=False)
        steady_step(mt + 1, (mt + 1) % 3, wait_next=False, issue_next=False,
                    last=(valid_last < chunk))

        wait_scatter((n_chunks - 2) % 3)
        wait_scatter((n_chunks - 1) % 3)

        plsc.subcore_barrier()
        out_r0 = c * n_pad + r0
        ddescs = []
        for zi in range(n_copy_full):
            ddescs.append(pltpu.async_copy(
                acc.at[pl.ds(r0 + zi * chunk, chunk)],
                out_hbm.at[pl.ds(out_r0 + zi * chunk, chunk)], sss0))
        for dd in ddescs:
            dd.wait()

    return sc_pass


_BLK = 1000


def _tc_pre1(x, W1, asmat, admat, R8):
    n, d_in = x.shape

    def body(x_ref, w_ref, as_ref, ad_ref, r8_ref, src_ref, dst_ref, self_ref):
        h = jnp.dot(x_ref[...], w_ref[...], preferred_element_type=jnp.float32)
        s = jnp.dot(h, as_ref[...], preferred_element_type=jnp.float32)
        d = jnp.dot(h, ad_ref[...], preferred_element_type=jnp.float32)
        t = s + d
        wself = jnp.exp(jnp.maximum(t, t * 0.2))
        src_ref[...] = jnp.concatenate([h, s, s], axis=1)
        dst_ref[...] = jnp.concatenate([d, d], axis=1)
        wexp = jnp.dot(wself, r8_ref[...], preferred_element_type=jnp.float32)
        self_ref[...] = jnp.concatenate([h * wexp, wself, wself], axis=1)

    return pl.pallas_call(
        body,
        grid=(n // _BLK,),
        in_specs=[
            pl.BlockSpec((_BLK, d_in), lambda i: (i, 0)),
            pl.BlockSpec((d_in, 64), lambda i: (0, 0)),
            pl.BlockSpec((64, 8), lambda i: (0, 0)),
            pl.BlockSpec((64, 8), lambda i: (0, 0)),
            pl.BlockSpec((8, 64), lambda i: (0, 0)),
        ],
        out_specs=[
            pl.BlockSpec((_BLK, 80), lambda i: (i, 0)),
            pl.BlockSpec((_BLK, 16), lambda i: (i, 0)),
            pl.BlockSpec((_BLK, 80), lambda i: (i, 0)),
        ],
        out_shape=[
            jax.ShapeDtypeStruct((n, 80), jnp.float32),
            jax.ShapeDtypeStruct((n, 16), jnp.float32),
            jax.ShapeDtypeStruct((n, 80), jnp.float32),
        ],
    )(x, W1, asmat, admat, R8)


def _tc_mid(acc1, self1, b1, W2, as2, ad2, R8):
    n = self1.shape[0]

    def body(acc_ref, self_ref, b1_ref, w2_ref, as2_ref, ad2_ref, r8_ref,
             src2_ref, dst2_ref, self2_ref):
        tot = acc_ref[0] + acc_ref[1] + self_ref[...]
        num = tot[:, 0:64]
        den = tot[:, 64:72]
        den_exp = jnp.dot(den, r8_ref[...], preferred_element_type=jnp.float32)
        h1 = num / (den_exp + 1e-16) + b1_ref[...]
        h1 = jnp.where(h1 > 0, h1, jnp.exp(jnp.minimum(h1, 0.0)) - 1.0)
        h2 = jnp.dot(h1, w2_ref[...], preferred_element_type=jnp.float32)
        s2 = jnp.sum(h2 * as2_ref[...], axis=1, keepdims=True)
        d2 = jnp.sum(h2 * ad2_ref[...], axis=1, keepdims=True)
        t2 = s2 + d2
        w2 = jnp.exp(jnp.maximum(t2, t2 * 0.2))
        src2_ref[...] = jnp.concatenate(
            [h2, jnp.broadcast_to(s2, (_BLK, 16))], axis=1)
        dst2_ref[...] = jnp.broadcast_to(d2, (_BLK, 16))
        self2_ref[...] = jnp.concatenate(
            [h2 * w2, jnp.broadcast_to(w2, (_BLK, 16))], axis=1)

    return pl.pallas_call(
        body,
        grid=(n // _BLK,),
        in_specs=[
            pl.BlockSpec((2, _BLK, 80), lambda i: (0, i, 0)),
            pl.BlockSpec((_BLK, 80), lambda i: (i, 0)),
            pl.BlockSpec((1, 64), lambda i: (0, 0)),
            pl.BlockSpec((64, 32), lambda i: (0, 0)),
            pl.BlockSpec((1, 32), lambda i: (0, 0)),
            pl.BlockSpec((1, 32), lambda i: (0, 0)),
            pl.BlockSpec((8, 64), lambda i: (0, 0)),
        ],
        out_specs=[
            pl.BlockSpec((_BLK, 48), lambda i: (i, 0)),
            pl.BlockSpec((_BLK, 16), lambda i: (i, 0)),
            pl.BlockSpec((_BLK, 48), lambda i: (i, 0)),
        ],
        out_shape=[
            jax.ShapeDtypeStruct((n, 48), jnp.float32),
            jax.ShapeDtypeStruct((n, 16), jnp.float32),
            jax.ShapeDtypeStruct((n, 48), jnp.float32),
        ],
    )(acc1, self1, b1, W2, as2, ad2, R8)


def _tc_post(acc2, self2, b2):
    n = self2.shape[0]

    def body(acc_ref, self_ref, b2_ref, out_ref):
        tot = acc_ref[0] + acc_ref[1] + self_ref[...]
        num = tot[:, 0:32]
        den = tot[:, 32:33]
        out_ref[...] = num / (den + 1e-16) + b2_ref[...]

    return pl.pallas_call(
        body,
        grid=(n // _BLK,),
        in_specs=[
            pl.BlockSpec((2, _BLK, 48), lambda i: (0, i, 0)),
            pl.BlockSpec((_BLK, 48), lambda i: (i, 0)),
            pl.BlockSpec((1, 32), lambda i: (0, 0)),
        ],
        out_specs=pl.BlockSpec((_BLK, 32), lambda i: (i, 0)),
        out_shape=jax.ShapeDtypeStruct((n, 32), jnp.float32),
    )(acc2, self2, b2)


def kernel(x, edge_index, W1, att_src1, att_dst1, b1,
           W2, att_src2, att_dst2, b2):
    n, d_in = x.shape
    n_edges = edge_index.shape[1]
    ei32 = edge_index.astype(jnp.int32)

    att_s1 = att_src1.reshape(64).astype(jnp.float32)
    att_d1 = att_dst1.reshape(64).astype(jnp.float32)
    R8 = jnp.kron(jnp.eye(8, dtype=jnp.float32),
                  jnp.ones((1, 8), jnp.float32))
    asmat = R8.T * att_s1[:, None]
    admat = R8.T * att_d1[:, None]

    srctab1, dsttab1, self1 = _tc_pre1(x, W1, asmat, admat, R8)

    chunk = 128
    n_pad = -(-n // (N_SUBCORES * chunk)) * chunk * N_SUBCORES
    ei32 = jnp.pad(ei32, ((0, 0), (0, chunk)))

    sc1 = _make_sc_edge_pass(n, n_edges, msg_w=64, chunk=chunk)
    acc1 = sc1(srctab1, dsttab1, ei32).reshape(2, n_pad, 80)[:, :n]

    srctab2, dsttab2, self2 = _tc_mid(
        acc1, self1, b1.reshape(1, 64), W2,
        att_src2.reshape(1, 32), att_dst2.reshape(1, 32), R8)

    sc2 = _make_sc_edge_pass(n, n_edges, msg_w=32, chunk=chunk)
    acc2 = sc2(srctab2, dsttab2, ei32).reshape(2, n_pad, 48)[:, :n]

    return _tc_post(acc2, self2, b2.reshape(1, 32))


# speedup vs baseline: 1.0150x; 1.0150x over previous
import functools

import jax
import jax.numpy as jnp
from jax import lax
from jax.experimental import pallas as pl
from jax.experimental.pallas import tpu as pltpu
from jax.experimental.pallas import tpu_sc as plsc

N_CORES = 2
N_SUBCORES = 16
LANES = 16

_GATHER_DN = lax.GatherDimensionNumbers(
    offset_dims=(), collapsed_slice_dims=(0,), start_index_map=(0,))
_UNROLL = 4


def _make_sc_edge_pass(n_nodes, n_edges, msg_w, chunk):
    row_w = msg_w + LANES
    expand = msg_w == 64
    n_workers = N_CORES * N_SUBCORES
    e_per_w = n_edges // n_workers
    assert e_per_w * n_workers == n_edges
    n_chunks = -(-e_per_w // chunk)
    valid_last = e_per_w - (n_chunks - 1) * chunk
    assert valid_last % LANES == 0
    rows_per_tile = -(-n_nodes // (N_SUBCORES * chunk)) * chunk
    n_pad = rows_per_tile * N_SUBCORES
    n_copy_full = rows_per_tile // chunk
    assert rows_per_tile % 8 == 0 and n_pad > n_nodes

    mesh = plsc.VectorSubcoreMesh(core_axis_name="c", subcore_axis_name="s")

    tail_peel = 2 + (n_chunks - 4) % 3
    n_triples = (n_chunks - 2 - tail_peel) // 3
    assert n_chunks >= 2 + tail_peel
    assert 2 + 3 * n_triples + tail_peel == n_chunks

    @functools.partial(
        pl.kernel,
        out_type=jax.ShapeDtypeStruct((N_CORES * n_pad, row_w), jnp.float32),
        mesh=mesh,
        compiler_params=pltpu.CompilerParams(use_tc_tiling_on_sc=False),
        scratch_types=(
            [pltpu.VMEM((2, chunk), jnp.int32)] * 3
            + [pltpu.VMEM((chunk,), jnp.int32)] * 3
            + [pltpu.VMEM((chunk, row_w), jnp.float32)] * 3
            + [pltpu.VMEM((chunk, LANES), jnp.float32)] * 3
            + [pltpu.VMEM_SHARED((n_pad, row_w), jnp.float32)]
            + [pltpu.SemaphoreType.DMA] * 12
        ),
    )
    def sc_pass(srctab, dsttab, ei_hbm, out_hbm,
                ij0, ij1, ij2,
                dscat0, dscat1, dscat2, A0, A1, A2, B0, B1, B2, acc,
                ssi0, ssi1, ssi2,
                sga0, sga1, sga2, sgb0, sgb1, sgb2, sss0, sss1, sss2):
        c = lax.axis_index("c")
        s = lax.axis_index("s")
        wid = c * N_SUBCORES + s

        ij = (ij0, ij1, ij2)
        dscat = (dscat0, dscat1, dscat2)
        A = (A0, A1, A2)
        B = (B0, B1, B2)
        ssi = (ssi0, ssi1, ssi2)
        sga = (sga0, sga1, sga2)
        sgb = (sgb0, sgb1, sgb2)
        sss = (sss0, sss1, sss2)

        zeros16 = jnp.zeros((LANES,), jnp.float32)

        def zero_row(e, carry):
            for q in range(row_w // LANES):
                A0[e, pl.ds(q * LANES, LANES)] = zeros16
            return carry

        lax.fori_loop(0, chunk, zero_row, 0)
        r0 = s * rows_per_tile
        zdescs = []
        for zi in range(n_copy_full):
            zdescs.append(pltpu.async_copy(
                A0, acc.at[pl.ds(r0 + zi * chunk, chunk)], sss0))
        for zd in zdescs:
            zd.wait()
        plsc.subcore_barrier()

        iota = lax.iota(jnp.int32, LANES)
        colpat = jnp.right_shift(iota, 3)

        ebase0 = wid * e_per_w

        def issue_idx(m, b):
            eb = ebase0 + m * chunk
            pltpu.async_copy(ei_hbm.at[:, pl.ds(eb, chunk)], ij[b], ssi[b])

        def wait_idx(b):
            pltpu.make_async_copy(
                ei_hbm.at[:, pl.ds(0, chunk)], ij[b], ssi[b]).wait()

        def issue_gather(b):
            pltpu.async_copy(srctab.at[ij[b].at[0]], A[b], sga[b])
            pltpu.async_copy(dsttab.at[ij[b].at[1]], B[b], sgb[b])

        def wait_gather(b):
            pltpu.make_async_copy(srctab.at[ij[b].at[0]], A[b], sga[b]).wait()
            pltpu.make_async_copy(dsttab.at[ij[b].at[1]], B[b], sgb[b]).wait()

        def issue_scatter(b):
            pltpu.async_copy(A[b], acc.at[dscat[b]], sss[b], add=True)

        def wait_scatter(b):
            pltpu.make_async_copy(A[b], acc.at[dscat[b]], sss[b]).wait()

        def compute(b):
            Ab, Bb = A[b], B[b]

            def one_edge(e):
                t = Ab[e, pl.ds(msg_w, LANES)] + Bb[e, pl.ds(0, LANES)]
                w = jnp.exp(jnp.maximum(t, t * 0.2))
                Ab[e, pl.ds(msg_w, LANES)] = w
                if expand:
                    for j in range(msg_w // LANES):
                        wj = lax.gather(
                            w, (colpat + (2 * j)).reshape(LANES, 1),
                            _GATHER_DN, slice_sizes=(1,),
                            mode=lax.GatherScatterMode.PROMISE_IN_BOUNDS)
                        Ab[e, pl.ds(LANES * j, LANES)] = (
                            Ab[e, pl.ds(LANES * j, LANES)] * wj)
                else:
                    for j in range(msg_w // LANES):
                        Ab[e, pl.ds(LANES * j, LANES)] = (
                            Ab[e, pl.ds(LANES * j, LANES)] * w)

            def edge_body(eg, icarry):
                for u in range(_UNROLL):
                    one_edge(eg * _UNROLL + u)
                return icarry

            lax.fori_loop(0, chunk // _UNROLL, edge_body, 0)

        def copy_dscat(b, last=False):
            for q in range(chunk // LANES):
                if last and q * LANES >= valid_last:
                    dscat[b][pl.ds(q * LANES, LANES)] = jnp.full(
                        (LANES,), n_pad - 1, jnp.int32)
                else:
                    dscat[b][pl.ds(q * LANES, LANES)] = (
                        ij[b][1, pl.ds(q * LANES, LANES)])

        def steady_step(m, b, first=False, wait_next=True, issue_next=True,
                        last=False):
            m = jnp.asarray(m, jnp.int32)
            bn = (b + 1) % 3
            if not first:
                wait_scatter(bn)
            if wait_next:
                wait_idx(bn)
                issue_gather(bn)
            wait_gather(b)
            copy_dscat(b, last=last)
            if issue_next:
                issue_idx(m + 2, (b + 2) % 3)
            compute(b)
            issue_scatter(b)

        issue_idx(0, 0)
        issue_idx(1, 1)
        wait_idx(0)
        issue_gather(0)
        steady_step(0, 0, first=True)
        steady_step(1, 1, first=True)

        def triple_body(tp, carry):
            steady_step(3 * tp + 2, 2)
            steady_step(3 * tp + 3, 0)
            steady_step(3 * tp + 4, 1)
            return carry

        lax.fori_loop(0, n_triples, triple_body, 0)

        for mt in range(n_chunks - tail_peel, n_chunks):
            steady_step(mt, mt % 3,
                        wait_next=(mt + 1 < n_chunks),
                        issue_next=(mt + 2 < n_chunks),
                        last=(mt == n_chunks - 1 and valid_last < chunk))

        wait_scatter((n_chunks - 2) % 3)
        wait_scatter((n_chunks - 1) % 3)

        plsc.subcore_barrier()
        out_r0 = c * n_pad + r0
        ddescs = []
        for zi in range(n_copy_full):
            ddescs.append(pltpu.async_copy(
                acc.at[pl.ds(r0 + zi * chunk, chunk)],
                out_hbm.at[pl.ds(out_r0 + zi * chunk, chunk)], sss0))
        for dd in ddescs:
            dd.wait()

    return sc_pass


_BLK = 1000


def _tc_pre1(x, W1, asmat, admat, R8):
    n, d_in = x.shape

    def body(x_ref, w_ref, as_ref, ad_ref, r8_ref, src_ref, dst_ref, self_ref):
        h = jnp.dot(x_ref[...], w_ref[...], preferred_element_type=jnp.float32)
        s = jnp.dot(h, as_ref[...], preferred_element_type=jnp.float32)
        d = jnp.dot(h, ad_ref[...], preferred_element_type=jnp.float32)
        t = s + d
        wself = jnp.exp(jnp.maximum(t, t * 0.2))
        src_ref[...] = jnp.concatenate([h, s, s], axis=1)
        dst_ref[...] = jnp.concatenate([d, d], axis=1)
        wexp = jnp.dot(wself, r8_ref[...], preferred_element_type=jnp.float32)
        self_ref[...] = jnp.concatenate([h * wexp, wself, wself], axis=1)

    return pl.pallas_call(
        body,
        grid=(n // _BLK,),
        in_specs=[
            pl.BlockSpec((_BLK, d_in), lambda i: (i, 0)),
            pl.BlockSpec((d_in, 64), lambda i: (0, 0)),
            pl.BlockSpec((64, 8), lambda i: (0, 0)),
            pl.BlockSpec((64, 8), lambda i: (0, 0)),
            pl.BlockSpec((8, 64), lambda i: (0, 0)),
        ],
        out_specs=[
            pl.BlockSpec((_BLK, 80), lambda i: (i, 0)),
            pl.BlockSpec((_BLK, 16), lambda i: (i, 0)),
            pl.BlockSpec((_BLK, 80), lambda i: (i, 0)),
        ],
        out_shape=[
            jax.ShapeDtypeStruct((n, 80), jnp.float32),
            jax.ShapeDtypeStruct((n, 16), jnp.float32),
            jax.ShapeDtypeStruct((n, 80), jnp.float32),
        ],
    )(x, W1, asmat, admat, R8)


def _tc_mid(acc1, self1, b1, W2, as2, ad2, R8):
    n = self1.shape[0]

    def body(acc_ref, self_ref, b1_ref, w2_ref, as2_ref, ad2_ref, r8_ref,
             src2_ref, dst2_ref, self2_ref):
        tot = acc_ref[0] + acc_ref[1] + self_ref[...]
        num = tot[:, 0:64]
        den = tot[:, 64:72]
        den_exp = jnp.dot(den, r8_ref[...], preferred_element_type=jnp.float32)
        h1 = num / (den_exp + 1e-16) + b1_ref[...]
        h1 = jnp.where(h1 > 0, h1, jnp.exp(jnp.minimum(h1, 0.0)) - 1.0)
        h2 = jnp.dot(h1, w2_ref[...], preferred_element_type=jnp.float32)
        s2 = jnp.sum(h2 * as2_ref[...], axis=1, keepdims=True)
        d2 = jnp.sum(h2 * ad2_ref[...], axis=1, keepdims=True)
        t2 = s2 + d2
        w2 = jnp.exp(jnp.maximum(t2, t2 * 0.2))
        src2_ref[...] = jnp.concatenate(
            [h2, jnp.broadcast_to(s2, (_BLK, 16))], axis=1)
        dst2_ref[...] = jnp.broadcast_to(d2, (_BLK, 16))
        self2_ref[...] = jnp.concatenate(
            [h2 * w2, jnp.broadcast_to(w2, (_BLK, 16))], axis=1)

    return pl.pallas_call(
        body,
        grid=(n // _BLK,),
        in_specs=[
            pl.BlockSpec((2, _BLK, 80), lambda i: (0, i, 0)),
            pl.BlockSpec((_BLK, 80), lambda i: (i, 0)),
            pl.BlockSpec((1, 64), lambda i: (0, 0)),
            pl.BlockSpec((64, 32), lambda i: (0, 0)),
            pl.BlockSpec((1, 32), lambda i: (0, 0)),
            pl.BlockSpec((1, 32), lambda i: (0, 0)),
            pl.BlockSpec((8, 64), lambda i: (0, 0)),
        ],
        out_specs=[
            pl.BlockSpec((_BLK, 48), lambda i: (i, 0)),
            pl.BlockSpec((_BLK, 16), lambda i: (i, 0)),
            pl.BlockSpec((_BLK, 48), lambda i: (i, 0)),
        ],
        out_shape=[
            jax.ShapeDtypeStruct((n, 48), jnp.float32),
            jax.ShapeDtypeStruct((n, 16), jnp.float32),
            jax.ShapeDtypeStruct((n, 48), jnp.float32),
        ],
    )(acc1, self1, b1, W2, as2, ad2, R8)


def _tc_post(acc2, self2, b2):
    n = self2.shape[0]

    def body(acc_ref, self_ref, b2_ref, out_ref):
        tot = acc_ref[0] + acc_ref[1] + self_ref[...]
        num = tot[:, 0:32]
        den = tot[:, 32:33]
        out_ref[...] = num / (den + 1e-16) + b2_ref[...]

    return pl.pallas_call(
        body,
        grid=(n // _BLK,),
        in_specs=[
            pl.BlockSpec((2, _BLK, 48), lambda i: (0, i, 0)),
            pl.BlockSpec((_BLK, 48), lambda i: (i, 0)),
            pl.BlockSpec((1, 32), lambda i: (0, 0)),
        ],
        out_specs=pl.BlockSpec((_BLK, 32), lambda i: (i, 0)),
        out_shape=jax.ShapeDtypeStruct((n, 32), jnp.float32),
    )(acc2, self2, b2)


def kernel(x, edge_index, W1, att_src1, att_dst1, b1,
           W2, att_src2, att_dst2, b2):
    n, d_in = x.shape
    n_edges = edge_index.shape[1]
    ei32 = edge_index.astype(jnp.int32)

    att_s1 = att_src1.reshape(64).astype(jnp.float32)
    att_d1 = att_dst1.reshape(64).astype(jnp.float32)
    R8 = jnp.kron(jnp.eye(8, dtype=jnp.float32),
                  jnp.ones((1, 8), jnp.float32))
    asmat = R8.T * att_s1[:, None]
    admat = R8.T * att_d1[:, None]

    srctab1, dsttab1, self1 = _tc_pre1(x, W1, asmat, admat, R8)

    chunk = 80
    n_pad = -(-n // (N_SUBCORES * chunk)) * chunk * N_SUBCORES
    e_per_w = n_edges // (N_CORES * N_SUBCORES)
    if (-(-e_per_w // chunk)) * chunk > e_per_w:
        ei32 = jnp.pad(ei32, ((0, 0), (0, chunk)))

    sc1 = _make_sc_edge_pass(n, n_edges, msg_w=64, chunk=chunk)
    acc1 = sc1(srctab1, dsttab1, ei32).reshape(2, n_pad, 80)[:, :n]

    srctab2, dsttab2, self2 = _tc_mid(
        acc1, self1, b1.reshape(1, 64), W2,
        att_src2.reshape(1, 32), att_dst2.reshape(1, 32), R8)

    sc2 = _make_sc_edge_pass(n, n_edges, msg_w=32, chunk=chunk)
    acc2 = sc2(srctab2, dsttab2, ei32).reshape(2, n_pad, 48)[:, :n]

    return _tc_post(acc2, self2, b2.reshape(1, 32))


# speedup vs baseline: 1.0474x; 1.0319x over previous
import functools

import jax
import jax.numpy as jnp
from jax import lax
from jax.experimental import pallas as pl
from jax.experimental.pallas import tpu as pltpu
from jax.experimental.pallas import tpu_sc as plsc

N_CORES = 2
N_SUBCORES = 16
LANES = 16

_GATHER_DN = lax.GatherDimensionNumbers(
    offset_dims=(), collapsed_slice_dims=(0,), start_index_map=(0,))
_UNROLL = 4


def _make_sc_edge_pass(n_nodes, n_edges, msg_w, chunk):
    row_w = msg_w + LANES
    expand = msg_w == 64
    n_workers = N_CORES * N_SUBCORES
    e_per_w = n_edges // n_workers
    assert e_per_w * n_workers == n_edges
    n_chunks = -(-e_per_w // chunk)
    valid_last = e_per_w - (n_chunks - 1) * chunk
    assert valid_last % LANES == 0
    rows_per_tile = -(-n_nodes // (N_SUBCORES * chunk)) * chunk
    n_pad = rows_per_tile * N_SUBCORES
    n_copy_full = rows_per_tile // chunk
    assert rows_per_tile % 8 == 0 and n_pad > n_nodes

    mesh = plsc.VectorSubcoreMesh(core_axis_name="c", subcore_axis_name="s")

    tail_peel = 2 + (n_chunks - 4) % 3
    n_triples = (n_chunks - 2 - tail_peel) // 3
    assert n_chunks >= 2 + tail_peel
    assert 2 + 3 * n_triples + tail_peel == n_chunks

    @functools.partial(
        pl.kernel,
        out_type=jax.ShapeDtypeStruct((N_CORES * n_pad, row_w), jnp.float32),
        mesh=mesh,
        compiler_params=pltpu.CompilerParams(use_tc_tiling_on_sc=False),
        scratch_types=(
            [pltpu.VMEM((2, chunk), jnp.int32)] * 3
            + [pltpu.VMEM((chunk,), jnp.int32)] * 3
            + [pltpu.VMEM((chunk, row_w), jnp.float32)] * 3
            + [pltpu.VMEM((chunk, LANES), jnp.float32)] * 3
            + [pltpu.VMEM_SHARED((n_pad, row_w), jnp.float32)]
            + [pltpu.SemaphoreType.DMA] * 12
        ),
    )
    def sc_pass(srctab, dsttab, ei_hbm, out_hbm,
                ij0, ij1, ij2,
                dscat0, dscat1, dscat2, A0, A1, A2, B0, B1, B2, acc,
                ssi0, ssi1, ssi2,
                sga0, sga1, sga2, sgb0, sgb1, sgb2, sss0, sss1, sss2):
        c = lax.axis_index("c")
        s = lax.axis_index("s")
        wid = c * N_SUBCORES + s

        ij = (ij0, ij1, ij2)
        dscat = (dscat0, dscat1, dscat2)
        A = (A0, A1, A2)
        B = (B0, B1, B2)
        ssi = (ssi0, ssi1, ssi2)
        sga = (sga0, sga1, sga2)
        sgb = (sgb0, sgb1, sgb2)
        sss = (sss0, sss1, sss2)

        zeros16 = jnp.zeros((LANES,), jnp.float32)

        def zero_row(e, carry):
            for q in range(row_w // LANES):
                A0[e, pl.ds(q * LANES, LANES)] = zeros16
            return carry

        lax.fori_loop(0, chunk, zero_row, 0)
        r0 = s * rows_per_tile
        zdescs = []
        for zi in range(n_copy_full):
            zdescs.append(pltpu.async_copy(
                A0, acc.at[pl.ds(r0 + zi * chunk, chunk)], sss0))
        for zd in zdescs:
            zd.wait()
        plsc.subcore_barrier()

        iota = lax.iota(jnp.int32, LANES)
        colpat = jnp.right_shift(iota, 3)

        ebase0 = wid * e_per_w

        def issue_idx(m, b):
            eb = ebase0 + m * chunk
            pltpu.async_copy(ei_hbm.at[:, pl.ds(eb, chunk)], ij[b], ssi[b])

        def wait_idx(b):
            pltpu.make_async_copy(
                ei_hbm.at[:, pl.ds(0, chunk)], ij[b], ssi[b]).wait()

        def issue_gather(b):
            pltpu.async_copy(srctab.at[ij[b].at[0]], A[b], sga[b])
            pltpu.async_copy(dsttab.at[ij[b].at[1]], B[b], sgb[b])

        def wait_gather(b):
            pltpu.make_async_copy(srctab.at[ij[b].at[0]], A[b], sga[b]).wait()
            pltpu.make_async_copy(dsttab.at[ij[b].at[1]], B[b], sgb[b]).wait()

        def issue_scatter(b):
            pltpu.async_copy(A[b], acc.at[dscat[b]], sss[b], add=True)

        def wait_scatter(b):
            pltpu.make_async_copy(A[b], acc.at[dscat[b]], sss[b]).wait()

        def compute(b):
            Ab, Bb = A[b], B[b]

            def one_edge(e):
                t = Ab[e, pl.ds(msg_w, LANES)] + Bb[e, pl.ds(0, LANES)]
                w = jnp.exp(jnp.maximum(t, t * 0.2))
                Ab[e, pl.ds(msg_w, LANES)] = w
                if expand:
                    for j in range(msg_w // LANES):
                        wj = lax.gather(
                            w, (colpat + (2 * j)).reshape(LANES, 1),
                            _GATHER_DN, slice_sizes=(1,),
                            mode=lax.GatherScatterMode.PROMISE_IN_BOUNDS)
                        Ab[e, pl.ds(LANES * j, LANES)] = (
                            Ab[e, pl.ds(LANES * j, LANES)] * wj)
                else:
                    for j in range(msg_w // LANES):
                        Ab[e, pl.ds(LANES * j, LANES)] = (
                            Ab[e, pl.ds(LANES * j, LANES)] * w)

            def edge_body(eg, icarry):
                for u in range(_UNROLL):
                    one_edge(eg * _UNROLL + u)
                return icarry

            lax.fori_loop(0, chunk // _UNROLL, edge_body, 0)

        def copy_dscat(b, last=False):
            for q in range(chunk // LANES):
                if last and q * LANES >= valid_last:
                    dscat[b][pl.ds(q * LANES, LANES)] = jnp.full(
                        (LANES,), n_pad - 1, jnp.int32)
                else:
                    dscat[b][pl.ds(q * LANES, LANES)] = (
                        ij[b][1, pl.ds(q * LANES, LANES)])

        def steady_step(m, b, first=False, wait_next=True, issue_next=True,
                        last=False):
            m = jnp.asarray(m, jnp.int32)
            bn = (b + 1) % 3
            if not first:
                wait_scatter(bn)
            if wait_next:
                wait_idx(bn)
                issue_gather(bn)
            wait_gather(b)
            copy_dscat(b, last=last)
            if issue_next:
                issue_idx(m + 2, (b + 2) % 3)
            compute(b)
            issue_scatter(b)

        issue_idx(0, 0)
        issue_idx(1, 1)
        wait_idx(0)
        issue_gather(0)
        steady_step(0, 0, first=True)
        steady_step(1, 1, first=True)

        def triple_body(tp, carry):
            steady_step(3 * tp + 2, 2)
            steady_step(3 * tp + 3, 0)
            steady_step(3 * tp + 4, 1)
            return carry

        lax.fori_loop(0, n_triples, triple_body, 0)

        for mt in range(n_chunks - tail_peel, n_chunks):
            steady_step(mt, mt % 3,
                        wait_next=(mt + 1 < n_chunks),
                        issue_next=(mt + 2 < n_chunks),
                        last=(mt == n_chunks - 1 and valid_last < chunk))

        wait_scatter((n_chunks - 2) % 3)
        wait_scatter((n_chunks - 1) % 3)

        plsc.subcore_barrier()
        out_r0 = c * n_pad + r0
        ddescs = []
        for zi in range(n_copy_full):
            ddescs.append(pltpu.async_copy(
                acc.at[pl.ds(r0 + zi * chunk, chunk)],
                out_hbm.at[pl.ds(out_r0 + zi * chunk, chunk)], sss0))
        for dd in ddescs:
            dd.wait()

    return sc_pass


_BLK = 1000


def _tc_pre1(x, W1, asmat, admat, R8):
    n, d_in = x.shape

    def body(x_ref, w_ref, as_ref, ad_ref, r8_ref, src_ref, dst_ref, self_ref):
        h = jnp.dot(x_ref[...], w_ref[...], preferred_element_type=jnp.float32)
        s = jnp.dot(h, as_ref[...], preferred_element_type=jnp.float32)
        d = jnp.dot(h, ad_ref[...], preferred_element_type=jnp.float32)
        t = s + d
        wself = jnp.exp(jnp.maximum(t, t * 0.2))
        src_ref[...] = jnp.concatenate([h, s, s], axis=1)
        dst_ref[...] = jnp.concatenate([d, d], axis=1)
        wexp = jnp.dot(wself, r8_ref[...], preferred_element_type=jnp.float32)
        self_ref[...] = jnp.concatenate([h * wexp, wself, wself], axis=1)

    return pl.pallas_call(
        body,
        grid=(n // _BLK,),
        in_specs=[
            pl.BlockSpec((_BLK, d_in), lambda i: (i, 0)),
            pl.BlockSpec((d_in, 64), lambda i: (0, 0)),
            pl.BlockSpec((64, 8), lambda i: (0, 0)),
            pl.BlockSpec((64, 8), lambda i: (0, 0)),
            pl.BlockSpec((8, 64), lambda i: (0, 0)),
        ],
        out_specs=[
            pl.BlockSpec((_BLK, 80), lambda i: (i, 0)),
            pl.BlockSpec((_BLK, 16), lambda i: (i, 0)),
            pl.BlockSpec((_BLK, 80), lambda i: (i, 0)),
        ],
        out_shape=[
            jax.ShapeDtypeStruct((n, 80), jnp.float32),
            jax.ShapeDtypeStruct((n, 16), jnp.float32),
            jax.ShapeDtypeStruct((n, 80), jnp.float32),
        ],
    )(x, W1, asmat, admat, R8)


def _tc_mid(acc1, self1, b1, W2, as2, ad2, R8):
    n = self1.shape[0]

    def body(acc0_ref, acc1_ref, self_ref, b1_ref, w2_ref, as2_ref, ad2_ref,
             r8_ref, src2_ref, dst2_ref, self2_ref):
        tot = acc0_ref[0] + acc1_ref[0] + self_ref[...]
        num = tot[:, 0:64]
        den = tot[:, 64:72]
        den_exp = jnp.dot(den, r8_ref[...], preferred_element_type=jnp.float32)
        h1 = num / (den_exp + 1e-16) + b1_ref[...]
        h1 = jnp.where(h1 > 0, h1, jnp.exp(jnp.minimum(h1, 0.0)) - 1.0)
        h2 = jnp.dot(h1, w2_ref[...], preferred_element_type=jnp.float32)
        s2 = jnp.sum(h2 * as2_ref[...], axis=1, keepdims=True)
        d2 = jnp.sum(h2 * ad2_ref[...], axis=1, keepdims=True)
        t2 = s2 + d2
        w2 = jnp.exp(jnp.maximum(t2, t2 * 0.2))
        src2_ref[...] = jnp.concatenate(
            [h2, jnp.broadcast_to(s2, (_BLK, 16))], axis=1)
        dst2_ref[...] = jnp.broadcast_to(d2, (_BLK, 16))
        self2_ref[...] = jnp.concatenate(
            [h2 * w2, jnp.broadcast_to(w2, (_BLK, 16))], axis=1)

    return pl.pallas_call(
        body,
        grid=(n // _BLK,),
        in_specs=[
            pl.BlockSpec((1, _BLK, 80), lambda i: (0, i, 0)),
            pl.BlockSpec((1, _BLK, 80), lambda i: (1, i, 0)),
            pl.BlockSpec((_BLK, 80), lambda i: (i, 0)),
            pl.BlockSpec((1, 64), lambda i: (0, 0)),
            pl.BlockSpec((64, 32), lambda i: (0, 0)),
            pl.BlockSpec((1, 32), lambda i: (0, 0)),
            pl.BlockSpec((1, 32), lambda i: (0, 0)),
            pl.BlockSpec((8, 64), lambda i: (0, 0)),
        ],
        out_specs=[
            pl.BlockSpec((_BLK, 48), lambda i: (i, 0)),
            pl.BlockSpec((_BLK, 16), lambda i: (i, 0)),
            pl.BlockSpec((_BLK, 48), lambda i: (i, 0)),
        ],
        out_shape=[
            jax.ShapeDtypeStruct((n, 48), jnp.float32),
            jax.ShapeDtypeStruct((n, 16), jnp.float32),
            jax.ShapeDtypeStruct((n, 48), jnp.float32),
        ],
    )(acc1, acc1, self1, b1, W2, as2, ad2, R8)


def _tc_post(acc2, self2, b2):
    n = self2.shape[0]

    def body(acc0_ref, acc1_ref, self_ref, b2_ref, out_ref):
        tot = acc0_ref[0] + acc1_ref[0] + self_ref[...]
        num = tot[:, 0:32]
        den = tot[:, 32:33]
        out_ref[...] = num / (den + 1e-16) + b2_ref[...]

    return pl.pallas_call(
        body,
        grid=(n // _BLK,),
        in_specs=[
            pl.BlockSpec((1, _BLK, 48), lambda i: (0, i, 0)),
            pl.BlockSpec((1, _BLK, 48), lambda i: (1, i, 0)),
            pl.BlockSpec((_BLK, 48), lambda i: (i, 0)),
            pl.BlockSpec((1, 32), lambda i: (0, 0)),
        ],
        out_specs=pl.BlockSpec((_BLK, 32), lambda i: (i, 0)),
        out_shape=jax.ShapeDtypeStruct((n, 32), jnp.float32),
    )(acc2, acc2, self2, b2)


def kernel(x, edge_index, W1, att_src1, att_dst1, b1,
           W2, att_src2, att_dst2, b2):
    n, d_in = x.shape
    n_edges = edge_index.shape[1]
    ei32 = edge_index.astype(jnp.int32)

    att_s1 = att_src1.reshape(64).astype(jnp.float32)
    att_d1 = att_dst1.reshape(64).astype(jnp.float32)
    R8 = jnp.kron(jnp.eye(8, dtype=jnp.float32),
                  jnp.ones((1, 8), jnp.float32))
    asmat = R8.T * att_s1[:, None]
    admat = R8.T * att_d1[:, None]

    srctab1, dsttab1, self1 = _tc_pre1(x, W1, asmat, admat, R8)

    chunk = 80
    n_pad = -(-n // (N_SUBCORES * chunk)) * chunk * N_SUBCORES
    e_per_w = n_edges // (N_CORES * N_SUBCORES)
    if (-(-e_per_w // chunk)) * chunk > e_per_w:
        ei32 = jnp.pad(ei32, ((0, 0), (0, chunk)))

    sc1 = _make_sc_edge_pass(n, n_edges, msg_w=64, chunk=chunk)
    acc1 = sc1(srctab1, dsttab1, ei32).reshape(2, n_pad, 80)

    srctab2, dsttab2, self2 = _tc_mid(
        acc1, self1, b1.reshape(1, 64), W2,
        att_src2.reshape(1, 32), att_dst2.reshape(1, 32), R8)

    sc2 = _make_sc_edge_pass(n, n_edges, msg_w=32, chunk=chunk)
    acc2 = sc2(srctab2, dsttab2, ei32).reshape(2, n_pad, 48)

    return _tc_post(acc2, self2, b2.reshape(1, 32))
